# baseline (speedup 1.0000x reference)
import jax
import jax.numpy as jnp
from jax.experimental import pallas as pl
from jax.experimental.pallas import tpu as pltpu

_BM = 512


def _fused_kernel(b_ref, w_ref, biast_ref, a_ref, outt_ref, ht_ref):
    @pl.when(pl.program_id(0) == 0)
    def _():
        ht_ref[...] = (
            jax.lax.dot_general(
                w_ref[...],
                b_ref[...],
                dimension_numbers=(((0,), (1,)), ((), ())),
                preferred_element_type=jnp.float32,
            )
            + biast_ref[...]
        )

    outt_ref[...] = jax.lax.dot_general(
        ht_ref[...],
        a_ref[...],
        dimension_numbers=(((1,), (1,)), ((), ())),
        preferred_element_type=jnp.float32,
    )


def kernel(a, b, W, bias):
    n, k = a.shape
    d_in = b.shape[1]
    d_out = W.shape[1]
    biast = bias.reshape(d_out, 1)

    outt = pl.pallas_call(
        _fused_kernel,
        grid=(n // _BM,),
        in_specs=[
            pl.BlockSpec((k, d_in), lambda i: (0, 0)),
            pl.BlockSpec((d_in, d_out), lambda i: (0, 0)),
            pl.BlockSpec((d_out, 1), lambda i: (0, 0)),
            pl.BlockSpec((_BM, k), lambda i: (i, 0)),
        ],
        out_specs=pl.BlockSpec((d_out, _BM), lambda i: (0, i)),
        out_shape=jax.ShapeDtypeStruct((d_out, n), jnp.float32),
        scratch_shapes=[pltpu.VMEM((d_out, k), jnp.float32)],
        compiler_params=pltpu.CompilerParams(
            dimension_semantics=("arbitrary",),
        ),
    )(b, W, biast, a)
    return outt.T
